# baseline (speedup 1.0000x reference)
import jax
import jax.numpy as jnp
from jax.experimental import pallas as pl

N = 10000
E = 160000
D_IN = 256
HID = 256
HEADS = 8
NPAD = 10240


def _mm_kernel(x_ref, w_ref, b_ref, o_ref):
    o_ref[...] = (
        jnp.dot(x_ref[...], w_ref[...], preferred_element_type=jnp.float32)
        + b_ref[...]
    )


def _fused_matmul(x, wcat, bcat, bm=512, bn=1024):
    m, k = x.shape
    f = wcat.shape[1]
    grid = (m // bm, f // bn)
    return pl.pallas_call(
        _mm_kernel,
        grid=grid,
        in_specs=[
            pl.BlockSpec((bm, k), lambda i, j: (i, 0)),
            pl.BlockSpec((k, bn), lambda i, j: (0, j)),
            pl.BlockSpec((1, bn), lambda i, j: (0, j)),
        ],
        out_specs=pl.BlockSpec((bm, bn), lambda i, j: (i, j)),
        out_shape=jax.ShapeDtypeStruct((m, f), jnp.float32),
    )(x, wcat, bcat.reshape(1, f))


def _edge_phase(q, k, v, edge_index, heads, out_ch, n):
    src = edge_index[0]
    dst = edge_index[1]
    q = q.reshape(n, heads, out_ch)
    k = k.reshape(n, heads, out_ch)
    v = v.reshape(n, heads, out_ch)
    logits = (q[dst] * k[src]).sum(-1) / jnp.sqrt(jnp.float32(out_ch))
    m = jax.ops.segment_max(logits, dst, num_segments=n)
    m = jnp.where(jnp.isfinite(m), m, 0.0)
    a = jnp.exp(logits - m[dst])
    denom = jax.ops.segment_sum(a, dst, num_segments=n)
    alpha = a / denom[dst]
    msg = alpha[:, :, None] * v[src]
    return jax.ops.segment_sum(msg, dst, num_segments=n).reshape(n, heads * out_ch)


def kernel(x, edge_index, Wq1, bq1, Wk1, bk1, Wv1, bv1, Ws1, bs1,
           Wq2, bq2, Wk2, bk2, Wv2, bv2, Ws2, bs2, Wl, bl):
    h1 = HEADS * HID
    xp = jnp.pad(x, ((0, NPAD - N), (0, 0)))
    w1 = jnp.concatenate([Wq1.T, Wk1.T, Wv1.T, Ws1.T], axis=1)
    b1 = jnp.concatenate([bq1, bk1, bv1, bs1])
    qkvs1 = _fused_matmul(xp, w1, b1)[:N]
    q1, k1, v1, s1 = (qkvs1[:, 0:h1], qkvs1[:, h1:2 * h1],
                      qkvs1[:, 2 * h1:3 * h1], qkvs1[:, 3 * h1:4 * h1])
    agg1 = _edge_phase(q1, k1, v1, edge_index, HEADS, HID, N)
    h = agg1 + s1

    hp = jnp.pad(h, ((0, NPAD - N), (0, 0)))
    w2 = jnp.concatenate([Wq2.T, Wk2.T, Wv2.T, Ws2.T], axis=1)
    b2 = jnp.concatenate([bq2, bk2, bv2, bs2])
    qkvs2 = _fused_matmul(hp, w2, b2, bm=512, bn=32)[:N]
    q2, k2, v2, s2 = (qkvs2[:, 0:8], qkvs2[:, 8:16],
                      qkvs2[:, 16:24], qkvs2[:, 24:32])
    agg2 = _edge_phase(q2, k2, v2, edge_index, HEADS, 1, N)
    h = agg2 + s2

    h = h @ Wl.T + bl
    h = jax.nn.relu(h)
    return h.mean(axis=0, keepdims=True)


# speedup vs baseline: 3.2982x; 3.2982x over previous
import functools
import math

import jax
import jax.numpy as jnp
from jax import lax
from jax.experimental import pallas as pl
from jax.experimental.pallas import tpu as pltpu
from jax.experimental.pallas import tpu_sc as plsc

N = 10000
E = 160000
D_IN = 256
HID = 256
HEADS = 8
NPAD = 10240
H1 = HEADS * HID

E_PAD = 163840
EPT_A = E_PAD // 32
GA = EPT_A // 16
W_ROWS = E_PAD + 16
DEN_ZR = NPAD // 16

EPT_B = E_PAD // 16
COLS = 128
NCHUNK = H1 // COLS
CPC = NCHUNK // 2
BB = 64
NBATCH = EPT_B // BB

HALF = NPAD // 2
ACC_ROWS = HALF + 128
ROWS_PER_TILE = HALF // 16
ZROWS = ACC_ROWS // 16
EB = 80
EPT_C = E // 16
GARBAGE = HALF


def _splat(v):
    return jnp.full((16,), v, jnp.int32)


def _perm_w(w):
    return w.reshape(HEADS, HID, -1).transpose(1, 0, 2).reshape(H1, -1)


def _perm_b(b):
    return b.reshape(HEADS, HID).T.reshape(-1)


def _mm_kernel(x_ref, w_ref, b_ref, o_ref):
    o_ref[...] = (
        jnp.dot(x_ref[...], w_ref[...], preferred_element_type=jnp.float32)
        + b_ref[...]
    )


def _proj_matmul(x, w, b, bm=512, bn=1024):
    m, k = x.shape
    f = w.shape[1]
    bn = min(bn, f)
    grid = (m // bm, f // bn)
    return pl.pallas_call(
        _mm_kernel,
        grid=grid,
        in_specs=[
            pl.BlockSpec((bm, k), lambda i, j: (i, 0)),
            pl.BlockSpec((k, bn), lambda i, j: (0, j)),
            pl.BlockSpec((1, bn), lambda i, j: (0, j)),
        ],
        out_specs=pl.BlockSpec((bm, bn), lambda i, j: (i, j)),
        out_shape=jax.ShapeDtypeStruct((m, f), jnp.float32),
    )(x, w, b.reshape(1, f))


def _proj_matmul_cc(x, w, b):
    return pl.pallas_call(
        _mm_kernel,
        grid=(NPAD // 512, NCHUNK),
        in_specs=[
            pl.BlockSpec((512, D_IN), lambda i, j: (i, 0)),
            pl.BlockSpec((D_IN, COLS), lambda i, j: (0, j)),
            pl.BlockSpec((1, COLS), lambda i, j: (0, j)),
        ],
        out_specs=pl.BlockSpec((512, COLS), lambda i, j: (j * (NPAD // 512) + i, 0)),
        out_shape=jax.ShapeDtypeStruct((NCHUNK * NPAD, COLS), jnp.float32),
    )(x, w, b.reshape(1, -1))


def _tc2_kernel(agg_ref, den_ref, s_ref, w_ref, b_ref, o_ref):
    j = pl.program_id(1)
    den8 = den_ref[...][:, 0:8]
    dex = jnp.broadcast_to(den8[:, None, :], (512, 16, 8)).reshape(512, 128)
    h = jnp.where(dex > 0, agg_ref[...] / dex, 0.0) + s_ref[...]
    part = jnp.dot(h, w_ref[...], preferred_element_type=jnp.float32)

    @pl.when(j == 0)
    def _():
        o_ref[...] = part + b_ref[...]

    @pl.when(j > 0)
    def _():
        o_ref[...] += part


def _tc3_kernel(acc_ref, qkvs2_ref, wl_ref, bl_ref, o_ref):
    i = pl.program_id(0)
    den = acc_ref[...][:, 0:8]
    agg = acc_ref[...][:, 8:16]
    h2 = jnp.where(den > 0, agg / den, 0.0) + qkvs2_ref[...][:, 24:32]
    z = jnp.dot(h2, wl_ref[...], preferred_element_type=jnp.float32) + bl_ref[...]
    z = jnp.maximum(z, 0.0)
    rows = i * 512 + lax.broadcasted_iota(jnp.int32, (512, 1), 0)
    z = jnp.where(rows < N, z, 0.0)
    part = jnp.sum(z, axis=0, keepdims=True)

    @pl.when(i == 0)
    def _():
        o_ref[...] = jnp.zeros_like(o_ref)

    o_ref[...] += part


def _tc3(accs, qkvs2, wlp, blp):
    return pl.pallas_call(
        _tc3_kernel,
        grid=(NPAD // 512,),
        in_specs=[
            pl.BlockSpec((512, 16), lambda i: (i, 0)),
            pl.BlockSpec((512, 32), lambda i: (i, 0)),
            pl.BlockSpec((8, 128), lambda i: (0, 0)),
            pl.BlockSpec((1, 128), lambda i: (0, 0)),
        ],
        out_specs=pl.BlockSpec((1, 128), lambda i: (0, 0)),
        out_shape=jax.ShapeDtypeStruct((1, 128), jnp.float32),
    )(accs, qkvs2, wlp, blp)


def _sca_body(q_hbm, k_hbm, srcp_hbm, dstp_hbm, w_hbm, den_hbm,
              den_sh, zb, srcb, dstb, qb, kb, wb, tmp, sem):
    c = lax.axis_index("c")
    s = lax.axis_index("s")

    def zr(i, _):
        zb[i, :] = jnp.zeros((16,), jnp.float32)
        return 0
    lax.fori_loop(0, DEN_ZR, zr, 0)
    pltpu.sync_copy(zb, den_sh.at[pl.ds(s * DEN_ZR, DEN_ZR)])
    plsc.subcore_barrier()

    base = (c * 16 + s) * EPT_A

    def grp(g, _):
        off = base + g * 16
        pltpu.sync_copy(srcp_hbm.at[pl.ds(off, 16)], srcb)
        pltpu.sync_copy(dstp_hbm.at[pl.ds(off, 16)], dstb)
        cq = pltpu.async_copy(q_hbm.at[dstb], qb, sem)
        ck = pltpu.async_copy(k_hbm.at[srcb], kb, sem)
        cq.wait()
        ck.wait()

        def edge(e, _):
            accs = [jnp.zeros((16,), jnp.float32) for _ in range(4)]
            for j in range(128):
                accs[j % 4] = accs[j % 4] + \
                    qb[e, pl.ds(j * 16, 16)] * kb[e, pl.ds(j * 16, 16)]
            a = (accs[0] + accs[1]) + (accs[2] + accs[3])
            tmp[pl.ds(0, 16)] = a
            tmp[pl.ds(16, 16)] = a
            wrow = tmp[pl.ds(0, 16)] + tmp[pl.ds(8, 16)]
            wb[e, :] = jnp.exp(wrow)
            return 0

        lax.fori_loop(0, 16, edge, 0)
        pltpu.sync_copy(wb, w_hbm.at[pl.ds(off, 16)])
        pltpu.sync_copy(wb, den_sh.at[dstb], add=True)
        return 0

    lax.fori_loop(0, GA, grp, 0)
    plsc.subcore_barrier()
    pltpu.sync_copy(
        den_sh.at[pl.ds(s * DEN_ZR, DEN_ZR)],
        den_hbm.at[pl.ds(c * NPAD + s * DEN_ZR, DEN_ZR)],
    )


_sc_a = functools.partial(
    pl.kernel,
    out_type=(
        jax.ShapeDtypeStruct((W_ROWS, 16), jnp.float32),
        jax.ShapeDtypeStruct((2 * NPAD, 16), jnp.float32),
    ),
    mesh=plsc.VectorSubcoreMesh(core_axis_name="c", subcore_axis_name="s"),
    compiler_params=pltpu.CompilerParams(use_tc_tiling_on_sc=False),
    scratch_types=[
        pltpu.VMEM_SHARED((NPAD, 16), jnp.float32),
        pltpu.VMEM((DEN_ZR, 16), jnp.float32),
        pltpu.VMEM((16,), jnp.int32),
        pltpu.VMEM((16,), jnp.int32),
        pltpu.VMEM((16, 2048), jnp.float32),
        pltpu.VMEM((16, 2048), jnp.float32),
        pltpu.VMEM((16, 16), jnp.float32),
        pltpu.VMEM((32,), jnp.float32),
        pltpu.SemaphoreType.DMA,
    ],
)(_sca_body)


def _scb_body(v_hbm, w_hbm, src_hbm, dst_hbm, agg_hbm,
              acc_sh, srcT, dstT, srcJ, vb, wb, sidx, didx, sem):
    c = lax.axis_index("c")
    s = lax.axis_index("s")
    gbase = s * EPT_B

    pltpu.sync_copy(src_hbm.at[pl.ds(gbase, EPT_B)], srcT)
    pltpu.sync_copy(dst_hbm.at[pl.ds(gbase, EPT_B)], dstT)

    def chunk(p, _):
        jj = c * CPC + p

        def zrow(i, _):
            for u in range(8):
                vb[i, pl.ds(u * 16, 16)] = jnp.zeros((16,), jnp.float32)
            return 0
        lax.fori_loop(0, BB, zrow, 0)
        for r in range(10):
            pltpu.sync_copy(vb, acc_sh.at[pl.ds(s * 640 + r * BB, BB)])

        def off(i, _):
            srcJ[pl.ds(i * 16, 16)] = \
                srcT[pl.ds(i * 16, 16)] + _splat(jj * NPAD)
            return 0
        lax.fori_loop(0, EPT_B // 16, off, 0)
        plsc.subcore_barrier()

        def batch(t, _):
            for u in range(BB // 16):
                sidx[pl.ds(u * 16, 16)] = srcJ[pl.ds(t * BB + u * 16, 16)]
                didx[pl.ds(u * 16, 16)] = dstT[pl.ds(t * BB + u * 16, 16)]
            cpv = pltpu.async_copy(v_hbm.at[sidx], vb, sem)
            cpw = pltpu.async_copy(
                w_hbm.at[pl.ds(gbase + t * BB, BB)], wb, sem)
            cpv.wait()
            cpw.wait()

            def edge(e, _):
                wrow = wb[e, :]
                for u in range(8):
                    vb[e, pl.ds(u * 16, 16)] = vb[e, pl.ds(u * 16, 16)] * wrow
                return 0
            lax.fori_loop(0, BB, edge, 0)
            pltpu.sync_copy(vb, acc_sh.at[didx], add=True)
            return 0
        lax.fori_loop(0, NBATCH, batch, 0)
        plsc.subcore_barrier()

        pltpu.sync_copy(
            acc_sh.at[pl.ds(s * 640, 640)],
            agg_hbm.at[pl.ds(jj * NPAD + s * 640, 640)],
        )
        plsc.subcore_barrier()
        return 0

    lax.fori_loop(0, CPC, chunk, 0)


_sc_b = functools.partial(
    pl.kernel,
    out_type=jax.ShapeDtypeStruct((NCHUNK * NPAD, COLS), jnp.float32),
    mesh=plsc.VectorSubcoreMesh(core_axis_name="c", subcore_axis_name="s"),
    compiler_params=pltpu.CompilerParams(use_tc_tiling_on_sc=False),
    scratch_types=[
        pltpu.VMEM_SHARED((NPAD, COLS), jnp.float32),
        pltpu.VMEM((EPT_B,), jnp.int32),
        pltpu.VMEM((EPT_B,), jnp.int32),
        pltpu.VMEM((EPT_B,), jnp.int32),
        pltpu.VMEM((BB, COLS), jnp.float32),
        pltpu.VMEM((BB, 16), jnp.float32),
        pltpu.VMEM((BB,), jnp.int32),
        pltpu.VMEM((BB,), jnp.int32),
        pltpu.SemaphoreType.DMA,
    ],
)(_scb_body)


def _scc_body(qd_hbm, kd_hbm, vt_hbm, src_hbm, dst_hbm, out_hbm,
              acc_sh, zbuf, srcb, dstb, dlocb, qb, kb, vb, cb, sem):
    c = lax.axis_index("c")
    s = lax.axis_index("s")
    lo = c * HALF

    def zb(i, _):
        zbuf[i, :] = jnp.zeros((16,), jnp.float32)
        return 0
    lax.fori_loop(0, ZROWS, zb, 0)
    pltpu.sync_copy(zbuf, acc_sh.at[pl.ds(s * ZROWS, ZROWS)])
    plsc.subcore_barrier()

    base = s * EPT_C

    def blk(b, _):
        off = base + b * EB
        pltpu.sync_copy(src_hbm.at[pl.ds(off, EB)], srcb)
        pltpu.sync_copy(dst_hbm.at[pl.ds(off, EB)], dstb)

        def idloop(i, _):
            d = dstb[pl.ds(i * 16, 16)]
            inr = (d >= lo) & (d < lo + HALF)
            dlocb[pl.ds(i * 16, 16)] = jnp.where(inr, d - lo, GARBAGE)
            return 0
        lax.fori_loop(0, EB // 16, idloop, 0)

        cq = pltpu.async_copy(qd_hbm.at[dstb], qb, sem)
        ck = pltpu.async_copy(kd_hbm.at[srcb], kb, sem)
        cv = pltpu.async_copy(vt_hbm.at[srcb], vb, sem)
        cq.wait()
        ck.wait()
        cv.wait()

        def eloop(e, _):
            w = jnp.exp(qb[e, :] * kb[e, :])
            cb[e, :] = w * vb[e, :]
            return 0
        lax.fori_loop(0, EB, eloop, 0)

        pltpu.sync_copy(cb, acc_sh.at[dlocb], add=True)
        return 0

    lax.fori_loop(0, EPT_C // EB, blk, 0)
    plsc.subcore_barrier()

    pltpu.sync_copy(
        acc_sh.at[pl.ds(s * ROWS_PER_TILE, ROWS_PER_TILE)],
        out_hbm.at[pl.ds(c * HALF + s * ROWS_PER_TILE, ROWS_PER_TILE)],
    )


_sc_c = functools.partial(
    pl.kernel,
    out_type=jax.ShapeDtypeStruct((NPAD, 16), jnp.float32),
    mesh=plsc.VectorSubcoreMesh(core_axis_name="c", subcore_axis_name="s"),
    compiler_params=pltpu.CompilerParams(use_tc_tiling_on_sc=False),
    scratch_types=[
        pltpu.VMEM_SHARED((ACC_ROWS, 16), jnp.float32),
        pltpu.VMEM((ZROWS, 16), jnp.float32),
        pltpu.VMEM((EB,), jnp.int32),
        pltpu.VMEM((EB,), jnp.int32),
        pltpu.VMEM((EB,), jnp.int32),
        pltpu.VMEM((EB, 16), jnp.float32),
        pltpu.VMEM((EB, 16), jnp.float32),
        pltpu.VMEM((EB, 16), jnp.float32),
        pltpu.VMEM((EB, 16), jnp.float32),
        pltpu.SemaphoreType.DMA,
    ],
)(_scc_body)


def kernel(x, edge_index, Wq1, bq1, Wk1, bk1, Wv1, bv1, Ws1, bs1,
           Wq2, bq2, Wk2, bk2, Wv2, bv2, Ws2, bs2, Wl, bl):
    src = edge_index[0]
    dst = edge_index[1]
    scale = 1.0 / math.sqrt(HID)
    xp = jnp.pad(x, ((0, NPAD - N), (0, 0)))

    q1 = _proj_matmul(xp, _perm_w(Wq1).T * scale, _perm_b(bq1) * scale)
    k1 = _proj_matmul(xp, _perm_w(Wk1).T, _perm_b(bk1))
    v1r = _proj_matmul_cc(xp, _perm_w(Wv1).T, _perm_b(bv1))
    s1r = _proj_matmul_cc(xp, _perm_w(Ws1).T, _perm_b(bs1))

    srcp = jnp.concatenate([src, jnp.zeros((E_PAD - E,), jnp.int32)])
    dstp = jnp.concatenate(
        [dst, jnp.full((E_PAD - E,), NPAD - 1, jnp.int32)])

    w1, denp = _sc_a(q1, k1, srcp, dstp)
    den1 = denp[:NPAD] + denp[NPAD:]

    agg1 = _sc_b(v1r, w1, srcp, dstp)

    w2 = jnp.concatenate([Wq2.T, Wk2.T, Wv2.T, Ws2.T], axis=1)
    w2 = w2.reshape(HEADS, HID, 32).transpose(1, 0, 2).reshape(H1, 32)
    b2 = jnp.concatenate([bq2, bk2, bv2, bs2])
    qkvs2 = pl.pallas_call(
        _tc2_kernel,
        grid=(NPAD // 512, NCHUNK),
        in_specs=[
            pl.BlockSpec((512, COLS), lambda i, j: (j * (NPAD // 512) + i, 0)),
            pl.BlockSpec((512, 16), lambda i, j: (i, 0)),
            pl.BlockSpec((512, COLS), lambda i, j: (j * (NPAD // 512) + i, 0)),
            pl.BlockSpec((COLS, 32), lambda i, j: (j, 0)),
            pl.BlockSpec((1, 32), lambda i, j: (0, 0)),
        ],
        out_specs=pl.BlockSpec((512, 32), lambda i, j: (i, 0)),
        out_shape=jax.ShapeDtypeStruct((NPAD, 32), jnp.float32),
    )(agg1, den1, s1r, w2, b2.reshape(1, 32))

    q2 = qkvs2[:, 0:8]
    k2 = qkvs2[:, 8:16]
    v2 = qkvs2[:, 16:24]
    qd = jnp.concatenate([q2, q2], axis=1)
    kd = jnp.concatenate([k2, k2], axis=1)
    vt = jnp.concatenate([jnp.ones_like(v2), v2], axis=1)
    accs = _sc_c(qd, kd, vt, src, dst)

    wlp = jnp.pad(Wl.T, ((0, 0), (0, 28)))
    blp = jnp.pad(bl, (0, 28)).reshape(1, 128)
    tot = _tc3(accs, qkvs2, wlp, blp)
    return tot[:, :100] / N


# speedup vs baseline: 4.9346x; 1.4961x over previous
import functools
import math

import jax
import jax.numpy as jnp
from jax import lax
from jax.experimental import pallas as pl
from jax.experimental.pallas import tpu as pltpu
from jax.experimental.pallas import tpu_sc as plsc

N = 10000
E = 160000
D_IN = 256
HID = 256
HEADS = 8
NPAD = 10240
H1 = HEADS * HID

E_PAD = 163840
EPT_A = E_PAD // 32
GA = EPT_A // 16
W_ROWS = E_PAD + 16
DEN_ZR = NPAD // 16

EPT_B = E_PAD // 16
COLS = 128
NCHUNK = H1 // COLS
CPC = NCHUNK // 2
BB = 64
NBATCH = EPT_B // BB

HALF = NPAD // 2
ACC_ROWS = HALF + 128
ROWS_PER_TILE = HALF // 16
ZROWS = ACC_ROWS // 16
EB = 80
EPT_C = E // 16
GARBAGE = HALF


def _splat(v):
    return jnp.full((16,), v, jnp.int32)


def _perm_w(w):
    return w.reshape(HEADS, HID, -1).transpose(1, 0, 2).reshape(H1, -1)


def _perm_b(b):
    return b.reshape(HEADS, HID).T.reshape(-1)


def _mm_kernel(x_ref, w_ref, b_ref, o_ref):
    o_ref[...] = (
        jnp.dot(x_ref[...], w_ref[...], preferred_element_type=jnp.float32)
        + b_ref[...]
    )


def _proj_matmul(x, w, b, bm=512, bn=1024):
    m, k = x.shape
    f = w.shape[1]
    bn = min(bn, f)
    grid = (m // bm, f // bn)
    return pl.pallas_call(
        _mm_kernel,
        grid=grid,
        in_specs=[
            pl.BlockSpec((bm, k), lambda i, j: (i, 0)),
            pl.BlockSpec((k, bn), lambda i, j: (0, j)),
            pl.BlockSpec((1, bn), lambda i, j: (0, j)),
        ],
        out_specs=pl.BlockSpec((bm, bn), lambda i, j: (i, j)),
        out_shape=jax.ShapeDtypeStruct((m, f), jnp.float32),
    )(x, w, b.reshape(1, f))


def _proj_matmul_cc(x, w, b, cols=COLS):
    nchunk = w.shape[1] // cols
    return pl.pallas_call(
        _mm_kernel,
        grid=(NPAD // 512, nchunk),
        in_specs=[
            pl.BlockSpec((512, D_IN), lambda i, j: (i, 0)),
            pl.BlockSpec((D_IN, cols), lambda i, j: (0, j)),
            pl.BlockSpec((1, cols), lambda i, j: (0, j)),
        ],
        out_specs=pl.BlockSpec(
            (512, cols), lambda i, j: (j * (NPAD // 512) + i, 0)),
        out_shape=jax.ShapeDtypeStruct((nchunk * NPAD, cols), jnp.float32),
    )(x, w, b.reshape(1, -1))


def _tc2_kernel(agg_ref, den_ref, s_ref, w_ref, b_ref, o_ref):
    j = pl.program_id(1)
    den8 = den_ref[...][:, 0:8]
    dex = jnp.broadcast_to(den8[:, None, :], (512, 16, 8)).reshape(512, 128)
    h = jnp.where(dex > 0, agg_ref[...] / dex, 0.0) + s_ref[...]
    part = jnp.dot(h, w_ref[...], preferred_element_type=jnp.float32)

    @pl.when(j == 0)
    def _():
        o_ref[...] = part + b_ref[...]

    @pl.when(j > 0)
    def _():
        o_ref[...] += part


def _tc3_kernel(acc_ref, qkvs2_ref, wl_ref, bl_ref, o_ref):
    i = pl.program_id(0)
    den = acc_ref[...][:, 0:8]
    agg = acc_ref[...][:, 8:16]
    h2 = jnp.where(den > 0, agg / den, 0.0) + qkvs2_ref[...][:, 24:32]
    z = jnp.dot(h2, wl_ref[...], preferred_element_type=jnp.float32) + bl_ref[...]
    z = jnp.maximum(z, 0.0)
    rows = i * 512 + lax.broadcasted_iota(jnp.int32, (512, 1), 0)
    z = jnp.where(rows < N, z, 0.0)
    part = jnp.sum(z, axis=0, keepdims=True)

    @pl.when(i == 0)
    def _():
        o_ref[...] = jnp.zeros_like(o_ref)

    o_ref[...] += part


def _tc3(accs, qkvs2, wlp, blp):
    return pl.pallas_call(
        _tc3_kernel,
        grid=(NPAD // 512,),
        in_specs=[
            pl.BlockSpec((512, 16), lambda i: (i, 0)),
            pl.BlockSpec((512, 32), lambda i: (i, 0)),
            pl.BlockSpec((8, 128), lambda i: (0, 0)),
            pl.BlockSpec((1, 128), lambda i: (0, 0)),
        ],
        out_specs=pl.BlockSpec((1, 128), lambda i: (0, 0)),
        out_shape=jax.ShapeDtypeStruct((1, 128), jnp.float32),
    )(accs, qkvs2, wlp, blp)


def _sca_body(q_hbm, k_hbm, srcp_hbm, dstp_hbm, w_hbm, den_hbm,
              den_sh, zb, srcT, dstT,
              qb0, qb1, kb0, kb1, qi0, qi1, ki0, ki1, didxb,
              pb, wb, tmp, gsem0, gsem1):
    c = lax.axis_index("c")
    s = lax.axis_index("s")
    qb = (qb0, qb1)
    kb = (kb0, kb1)
    qi = (qi0, qi1)
    ki = (ki0, ki1)
    gsem = (gsem0, gsem1)

    def zr(i, _):
        zb[i, :] = jnp.zeros((16,), jnp.float32)
        return 0
    lax.fori_loop(0, DEN_ZR, zr, 0)
    pltpu.sync_copy(zb, den_sh.at[pl.ds(s * DEN_ZR, DEN_ZR)])
    plsc.subcore_barrier()

    base = (c * 16 + s) * EPT_A
    pltpu.sync_copy(srcp_hbm.at[pl.ds(base, EPT_A)], srcT)
    pltpu.sync_copy(dstp_hbm.at[pl.ds(base, EPT_A)], dstT)

    def prep(g, ph, sl):
        o = _splat(ph * NPAD)
        qi[sl][pl.ds(0, 16)] = dstT[pl.ds(g * 16, 16)] + o
        ki[sl][pl.ds(0, 16)] = srcT[pl.ds(g * 16, 16)] + o

    def fire_gather(sl):
        pltpu.async_copy(q_hbm.at[qi[sl]], qb[sl], gsem[sl])
        pltpu.async_copy(k_hbm.at[ki[sl]], kb[sl], gsem[sl])

    def wait_gather(sl):
        pltpu.make_async_copy(q_hbm.at[qi[sl]], qb[sl], gsem[sl]).wait()
        pltpu.make_async_copy(k_hbm.at[ki[sl]], kb[sl], gsem[sl]).wait()

    def fold(sl, e):
        accs = [jnp.zeros((16,), jnp.float32) for _ in range(4)]
        for j in range(64):
            accs[j % 4] = accs[j % 4] + \
                qb[sl][e, pl.ds(j * 16, 16)] * kb[sl][e, pl.ds(j * 16, 16)]
        a = (accs[0] + accs[1]) + (accs[2] + accs[3])
        tmp[pl.ds(0, 16)] = a
        tmp[pl.ds(16, 16)] = a
        return tmp[pl.ds(0, 16)] + tmp[pl.ds(8, 16)]

    def grp(i, _):
        off = base + i * 16
        wait_gather(0)
        prep(i, 1, 1)
        fire_gather(1)

        def edge0(e, _):
            pb[e, :] = fold(0, e)
            return 0
        lax.fori_loop(0, 16, edge0, 0)

        wait_gather(1)
        prep(jnp.minimum(i + 1, GA - 1), 0, 0)
        fire_gather(0)

        def edge1(e, _):
            wb[e, :] = jnp.exp(pb[e, :] + fold(1, e))
            return 0
        lax.fori_loop(0, 16, edge1, 0)

        didxb[pl.ds(0, 16)] = dstT[pl.ds(i * 16, 16)]
        pltpu.sync_copy(wb, w_hbm.at[pl.ds(off, 16)])
        pltpu.sync_copy(wb, den_sh.at[didxb], add=True)
        return 0

    prep(0, 0, 0)
    fire_gather(0)
    lax.fori_loop(0, GA, grp, 0)
    wait_gather(0)
    plsc.subcore_barrier()
    pltpu.sync_copy(
        den_sh.at[pl.ds(s * DEN_ZR, DEN_ZR)],
        den_hbm.at[pl.ds(c * NPAD + s * DEN_ZR, DEN_ZR)],
    )


_sc_a = functools.partial(
    pl.kernel,
    out_type=(
        jax.ShapeDtypeStruct((W_ROWS, 16), jnp.float32),
        jax.ShapeDtypeStruct((2 * NPAD, 16), jnp.float32),
    ),
    mesh=plsc.VectorSubcoreMesh(core_axis_name="c", subcore_axis_name="s"),
    compiler_params=pltpu.CompilerParams(use_tc_tiling_on_sc=False),
    scratch_types=[
        pltpu.VMEM_SHARED((NPAD, 16), jnp.float32),
        pltpu.VMEM((DEN_ZR, 16), jnp.float32),
        pltpu.VMEM((EPT_A,), jnp.int32),
        pltpu.VMEM((EPT_A,), jnp.int32),
        pltpu.VMEM((16, 1024), jnp.float32),
        pltpu.VMEM((16, 1024), jnp.float32),
        pltpu.VMEM((16, 1024), jnp.float32),
        pltpu.VMEM((16, 1024), jnp.float32),
        pltpu.VMEM((16,), jnp.int32),
        pltpu.VMEM((16,), jnp.int32),
        pltpu.VMEM((16,), jnp.int32),
        pltpu.VMEM((16,), jnp.int32),
        pltpu.VMEM((16,), jnp.int32),
        pltpu.VMEM((16, 16), jnp.float32),
        pltpu.VMEM((16, 16), jnp.float32),
        pltpu.VMEM((32,), jnp.float32),
        pltpu.SemaphoreType.DMA,
        pltpu.SemaphoreType.DMA,
    ],
)(_sca_body)


def _scb_body(v_hbm, w_hbm, src_hbm, dst_hbm, agg_hbm,
              acc_sh, srcT, dstT,
              vb0, vb1, wb0, wb1, sidx0, sidx1, didx0, didx1,
              gsem0, gsem1):
    c = lax.axis_index("c")
    s = lax.axis_index("s")
    gbase = s * EPT_B
    vb = (vb0, vb1)
    wb = (wb0, wb1)
    sidx = (sidx0, sidx1)
    didx = (didx0, didx1)
    gsem = (gsem0, gsem1)

    pltpu.sync_copy(src_hbm.at[pl.ds(gbase, EPT_B)], srcT)
    pltpu.sync_copy(dst_hbm.at[pl.ds(gbase, EPT_B)], dstT)

    def fire_gather(g, sl):
        pltpu.async_copy(v_hbm.at[sidx[sl]], vb[sl], gsem[sl])
        pltpu.async_copy(w_hbm.at[pl.ds(gbase + g * BB, BB)], wb[sl], gsem[sl])

    def wait_gather(sl):
        pltpu.make_async_copy(v_hbm.at[sidx[sl]], vb[sl], gsem[sl]).wait()
        pltpu.make_async_copy(
            w_hbm.at[pl.ds(gbase, BB)], wb[sl], gsem[sl]).wait()

    def compute(sl):
        def edge(e, _):
            wrow = wb[sl][e, :]
            for u in range(8):
                vb[sl][e, pl.ds(u * 16, 16)] = \
                    vb[sl][e, pl.ds(u * 16, 16)] * wrow
            return 0
        lax.fori_loop(0, BB, edge, 0)

    def chunk(p, _):
        jj = c * CPC + p

        def zrow(i, _):
            for u in range(8):
                vb0[i, pl.ds(u * 16, 16)] = jnp.zeros((16,), jnp.float32)
            return 0
        lax.fori_loop(0, BB, zrow, 0)
        for r in range(10):
            pltpu.sync_copy(vb0, acc_sh.at[pl.ds(s * 640 + r * BB, BB)])

        def prep(g, sl):
            o = _splat(jj * NPAD)
            for u in range(BB // 16):
                sidx[sl][pl.ds(u * 16, 16)] = \
                    srcT[pl.ds(g * BB + u * 16, 16)] + o
                didx[sl][pl.ds(u * 16, 16)] = dstT[pl.ds(g * BB + u * 16, 16)]
        plsc.subcore_barrier()

        prep(0, 0)
        fire_gather(0, 0)

        def pair(i, _):
            g0 = i * 2
            wait_gather(0)
            prep(g0 + 1, 1)
            fire_gather(g0 + 1, 1)
            compute(0)
            pltpu.sync_copy(vb[0], acc_sh.at[didx[0]], add=True)
            wait_gather(1)
            gnext = jnp.minimum(g0 + 2, NBATCH - 2)
            prep(gnext, 0)
            fire_gather(gnext, 0)
            compute(1)
            pltpu.sync_copy(vb[1], acc_sh.at[didx[1]], add=True)
            return 0

        lax.fori_loop(0, NBATCH // 2, pair, 0)
        wait_gather(0)
        plsc.subcore_barrier()

        pltpu.sync_copy(
            acc_sh.at[pl.ds(s * 640, 640)],
            agg_hbm.at[pl.ds(jj * NPAD + s * 640, 640)],
        )
        plsc.subcore_barrier()
        return 0

    lax.fori_loop(0, CPC, chunk, 0)


_sc_b = functools.partial(
    pl.kernel,
    out_type=jax.ShapeDtypeStruct((NCHUNK * NPAD, COLS), jnp.float32),
    mesh=plsc.VectorSubcoreMesh(core_axis_name="c", subcore_axis_name="s"),
    compiler_params=pltpu.CompilerParams(use_tc_tiling_on_sc=False),
    scratch_types=[
        pltpu.VMEM_SHARED((NPAD, COLS), jnp.float32),
        pltpu.VMEM((EPT_B,), jnp.int32),
        pltpu.VMEM((EPT_B,), jnp.int32),
        pltpu.VMEM((BB, COLS), jnp.float32),
        pltpu.VMEM((BB, COLS), jnp.float32),
        pltpu.VMEM((BB, 16), jnp.float32),
        pltpu.VMEM((BB, 16), jnp.float32),
        pltpu.VMEM((BB,), jnp.int32),
        pltpu.VMEM((BB,), jnp.int32),
        pltpu.VMEM((BB,), jnp.int32),
        pltpu.VMEM((BB,), jnp.int32),
        pltpu.SemaphoreType.DMA,
        pltpu.SemaphoreType.DMA,
    ],
)(_scb_body)


def _scc_body(qd_hbm, kd_hbm, vt_hbm, src_hbm, dst_hbm, out_hbm,
              acc_sh, zbuf, srcb, dstb, dlocb, qb, kb, vb, cb, sem):
    c = lax.axis_index("c")
    s = lax.axis_index("s")
    lo = c * HALF

    def zb(i, _):
        zbuf[i, :] = jnp.zeros((16,), jnp.float32)
        return 0
    lax.fori_loop(0, ZROWS, zb, 0)
    pltpu.sync_copy(zbuf, acc_sh.at[pl.ds(s * ZROWS, ZROWS)])
    plsc.subcore_barrier()

    base = s * EPT_C

    def blk(b, _):
        off = base + b * EB
        pltpu.sync_copy(src_hbm.at[pl.ds(off, EB)], srcb)
        pltpu.sync_copy(dst_hbm.at[pl.ds(off, EB)], dstb)

        def idloop(i, _):
            d = dstb[pl.ds(i * 16, 16)]
            inr = (d >= lo) & (d < lo + HALF)
            dlocb[pl.ds(i * 16, 16)] = jnp.where(inr, d - lo, GARBAGE)
            return 0
        lax.fori_loop(0, EB // 16, idloop, 0)

        cq = pltpu.async_copy(qd_hbm.at[dstb], qb, sem)
        ck = pltpu.async_copy(kd_hbm.at[srcb], kb, sem)
        cv = pltpu.async_copy(vt_hbm.at[srcb], vb, sem)
        cq.wait()
        ck.wait()
        cv.wait()

        def eloop(e, _):
            w = jnp.exp(qb[e, :] * kb[e, :])
            cb[e, :] = w * vb[e, :]
            return 0
        lax.fori_loop(0, EB, eloop, 0)

        pltpu.sync_copy(cb, acc_sh.at[dlocb], add=True)
        return 0

    lax.fori_loop(0, EPT_C // EB, blk, 0)
    plsc.subcore_barrier()

    pltpu.sync_copy(
        acc_sh.at[pl.ds(s * ROWS_PER_TILE, ROWS_PER_TILE)],
        out_hbm.at[pl.ds(c * HALF + s * ROWS_PER_TILE, ROWS_PER_TILE)],
    )


_sc_c = functools.partial(
    pl.kernel,
    out_type=jax.ShapeDtypeStruct((NPAD, 16), jnp.float32),
    mesh=plsc.VectorSubcoreMesh(core_axis_name="c", subcore_axis_name="s"),
    compiler_params=pltpu.CompilerParams(use_tc_tiling_on_sc=False),
    scratch_types=[
        pltpu.VMEM_SHARED((ACC_ROWS, 16), jnp.float32),
        pltpu.VMEM((ZROWS, 16), jnp.float32),
        pltpu.VMEM((EB,), jnp.int32),
        pltpu.VMEM((EB,), jnp.int32),
        pltpu.VMEM((EB,), jnp.int32),
        pltpu.VMEM((EB, 16), jnp.float32),
        pltpu.VMEM((EB, 16), jnp.float32),
        pltpu.VMEM((EB, 16), jnp.float32),
        pltpu.VMEM((EB, 16), jnp.float32),
        pltpu.SemaphoreType.DMA,
    ],
)(_scc_body)


def kernel(x, edge_index, Wq1, bq1, Wk1, bk1, Wv1, bv1, Ws1, bs1,
           Wq2, bq2, Wk2, bk2, Wv2, bv2, Ws2, bs2, Wl, bl):
    src = edge_index[0]
    dst = edge_index[1]
    scale = 1.0 / math.sqrt(HID)
    xp = jnp.pad(x, ((0, NPAD - N), (0, 0)))

    q1 = _proj_matmul_cc(xp, _perm_w(Wq1).T * scale, _perm_b(bq1) * scale,
                         cols=1024)
    k1 = _proj_matmul_cc(xp, _perm_w(Wk1).T, _perm_b(bk1), cols=1024)
    v1r = _proj_matmul_cc(xp, _perm_w(Wv1).T, _perm_b(bv1))
    s1r = _proj_matmul_cc(xp, _perm_w(Ws1).T, _perm_b(bs1))

    srcp = jnp.concatenate([src, jnp.zeros((E_PAD - E,), jnp.int32)])
    dstp = jnp.concatenate(
        [dst, jnp.full((E_PAD - E,), NPAD - 1, jnp.int32)])

    w1, denp = _sc_a(q1, k1, srcp, dstp)
    den1 = denp[:NPAD] + denp[NPAD:]

    agg1 = _sc_b(v1r, w1, srcp, dstp)

    w2 = jnp.concatenate([Wq2.T, Wk2.T, Wv2.T, Ws2.T], axis=1)
    w2 = w2.reshape(HEADS, HID, 32).transpose(1, 0, 2).reshape(H1, 32)
    b2 = jnp.concatenate([bq2, bk2, bv2, bs2])
    qkvs2 = pl.pallas_call(
        _tc2_kernel,
        grid=(NPAD // 512, NCHUNK),
        in_specs=[
            pl.BlockSpec((512, COLS), lambda i, j: (j * (NPAD // 512) + i, 0)),
            pl.BlockSpec((512, 16), lambda i, j: (i, 0)),
            pl.BlockSpec((512, COLS), lambda i, j: (j * (NPAD // 512) + i, 0)),
            pl.BlockSpec((COLS, 32), lambda i, j: (j, 0)),
            pl.BlockSpec((1, 32), lambda i, j: (0, 0)),
        ],
        out_specs=pl.BlockSpec((512, 32), lambda i, j: (i, 0)),
        out_shape=jax.ShapeDtypeStruct((NPAD, 32), jnp.float32),
    )(agg1, den1, s1r, w2, b2.reshape(1, 32))

    q2 = qkvs2[:, 0:8]
    k2 = qkvs2[:, 8:16]
    v2 = qkvs2[:, 16:24]
    qd = jnp.concatenate([q2, q2], axis=1)
    kd = jnp.concatenate([k2, k2], axis=1)
    vt = jnp.concatenate([jnp.ones_like(v2), v2], axis=1)
    accs = _sc_c(qd, kd, vt, src, dst)

    wlp = jnp.pad(Wl.T, ((0, 0), (0, 28)))
    blp = jnp.pad(bl, (0, 28)).reshape(1, 128)
    tot = _tc3(accs, qkvs2, wlp, blp)
    return tot[:, :100] / N
